# Initial kernel scaffold; baseline (speedup 1.0000x reference)
import jax
import jax.numpy as jnp
from jax.experimental import pallas as pl


def kernel(embeddings, indices):
    raise NotImplementedError("write your pallas kernel here")


# speedup vs baseline: 3.0210x; 3.0210x over previous
import functools

import jax
import jax.numpy as jnp
from jax import lax
from jax.experimental import pallas as pl
from jax.experimental.pallas import tpu as pltpu
from jax.experimental.pallas import tpu_sc as plsc


def _make_sc_kernel(V, D, N, chunk_rows, idx_row):
    info = plsc.get_sparse_core_info()
    NC, NS, L = info.num_cores, info.num_subcores, info.num_lanes
    NW = NC * NS
    assert D % L == 0 and N % (NW * chunk_rows) == 0 and chunk_rows % idx_row == 0
    n_chunks = N // (NW * chunk_rows)
    g_streams = chunk_rows // idx_row
    n_groups = chunk_rows // L
    idx_rows_per_worker = N // (NW * idx_row)

    mesh = plsc.VectorSubcoreMesh(core_axis_name="c", subcore_axis_name="s")

    @functools.partial(
        pl.kernel,
        mesh=mesh,
        out_type=jax.ShapeDtypeStruct((N, D), jnp.float32),
        scratch_types=[
            pltpu.VMEM((idx_rows_per_worker, idx_row), jnp.int32),
            pltpu.VMEM((chunk_rows, D), jnp.float32),
            pltpu.SemaphoreType.DMA,
        ],
        compiler_params=pltpu.CompilerParams(
            needs_layout_passes=False, use_tc_tiling_on_sc=False
        ),
    )
    def sc_kernel(emb_hbm, idx_hbm, out_hbm, idx_v, rows_v, sem):
        wid = lax.axis_index("s") * NC + lax.axis_index("c")
        lane = lax.iota(jnp.int32, L)
        zero16 = jnp.zeros((L,), jnp.int32)
        pltpu.sync_copy(idx_hbm.at[wid], idx_v)

        def chunk_body(k, carry):
            copies = [
                pltpu.async_copy(
                    emb_hbm.at[idx_v.at[k * g_streams + j]],
                    rows_v.at[pl.ds(j * idx_row, idx_row)],
                    sem,
                )
                for j in range(g_streams)
            ]
            for c in copies:
                c.wait()

            def grp_body(g, carry2):
                row_ids = g * L + lane
                acc = jnp.zeros((L,), jnp.float32)
                for col in range(1, D):
                    cvec = jnp.full((L,), col, jnp.int32)
                    v = plsc.load_gather(rows_v, [row_ids, cvec])
                    acc = acc + v * v
                x = jnp.maximum(acc + 1.0, 1.0 + 1e-12)
                i = lax.bitcast_convert_type(x, jnp.int32)
                i = 0x5F3759DF - lax.shift_right_arithmetic(i, 1)
                y = lax.bitcast_convert_type(i, jnp.float32)
                y = y * (1.5 - 0.5 * x * y * y)
                y = y * (1.5 - 0.5 * x * y * y)
                y = y * (1.5 - 0.5 * x * y * y)
                x0 = x * y
                plsc.store_scatter(rows_v, [row_ids, zero16], x0)
                return carry2

            lax.fori_loop(0, n_groups, grp_body, 0)
            out_start = (wid * n_chunks + k) * chunk_rows
            pltpu.sync_copy(rows_v, out_hbm.at[pl.ds(out_start, chunk_rows)])
            return carry

        lax.fori_loop(0, n_chunks, chunk_body, 0)

    return sc_kernel


def kernel(embeddings, indices):
    V, D = embeddings.shape
    N = indices.size
    idx_row = 128
    chunk_rows = 640
    nw = 32
    idx3d = indices.reshape(nw, N // (nw * idx_row), idx_row).astype(jnp.int32)
    sc = _make_sc_kernel(V, D, N, chunk_rows, idx_row)
    out = sc(embeddings, idx3d)
    return out.reshape(indices.shape + (D,))


# speedup vs baseline: 3.1461x; 1.0414x over previous
import functools

import jax
import jax.numpy as jnp
from jax import lax
from jax.experimental import pallas as pl
from jax.experimental.pallas import tpu as pltpu
from jax.experimental.pallas import tpu_sc as plsc


def _make_sc_kernel(V, D, N, chunk_rows, idx_row):
    info = plsc.get_sparse_core_info()
    NC, NS, L = info.num_cores, info.num_subcores, info.num_lanes
    NW = NC * NS
    assert D % L == 0 and N % (NW * chunk_rows) == 0 and chunk_rows % idx_row == 0
    n_chunks = N // (NW * chunk_rows)
    g_streams = chunk_rows // idx_row
    n_groups = chunk_rows // L
    idx_rows_per_worker = N // (NW * idx_row)

    mesh = plsc.VectorSubcoreMesh(core_axis_name="c", subcore_axis_name="s")

    @functools.partial(
        pl.kernel,
        mesh=mesh,
        out_type=jax.ShapeDtypeStruct((N, D), jnp.float32),
        scratch_types=[
            pltpu.VMEM((idx_rows_per_worker, idx_row), jnp.int32),
            pltpu.VMEM((chunk_rows, D), jnp.float32),
            pltpu.VMEM((chunk_rows, D), jnp.float32),
            pltpu.SemaphoreType.DMA,
            pltpu.SemaphoreType.DMA,
            pltpu.SemaphoreType.DMA,
            pltpu.SemaphoreType.DMA,
        ],
        compiler_params=pltpu.CompilerParams(
            needs_layout_passes=False, use_tc_tiling_on_sc=False
        ),
    )
    def sc_kernel(emb_hbm, idx_hbm, out_hbm, idx_v, rows_a, rows_b,
                  gsem_a, gsem_b, osem_a, osem_b):
        wid = lax.axis_index("s") * NC + lax.axis_index("c")
        lane = lax.iota(jnp.int32, L)
        zero16 = jnp.zeros((L,), jnp.int32)
        pltpu.sync_copy(idx_hbm.at[wid], idx_v)

        bufs = (rows_a, rows_b)
        gsems = (gsem_a, gsem_b)
        osems = (osem_a, osem_b)

        def fire_gathers(k, buf, gsem):
            return [
                pltpu.async_copy(
                    emb_hbm.at[idx_v.at[k * g_streams + j]],
                    buf.at[pl.ds(j * idx_row, idx_row)],
                    gsem,
                )
                for j in range(g_streams)
            ]

        def fixup_chunk(buf):
            def grp_body(g, carry2):
                row_ids = g * L + lane
                acc = jnp.zeros((L,), jnp.float32)
                for col in range(1, D):
                    cvec = jnp.full((L,), col, jnp.int32)
                    v = plsc.load_gather(buf, [row_ids, cvec])
                    acc = acc + v * v
                x = jnp.maximum(acc + 1.0, 1.0 + 1e-12)
                i = lax.bitcast_convert_type(x, jnp.int32)
                i = 0x5F3759DF - lax.shift_right_arithmetic(i, 1)
                y = lax.bitcast_convert_type(i, jnp.float32)
                y = y * (1.5 - 0.5 * x * y * y)
                y = y * (1.5 - 0.5 * x * y * y)
                y = y * (1.5 - 0.5 * x * y * y)
                x0 = x * y
                plsc.store_scatter(buf, [row_ids, zero16], x0)
                return carry2

            lax.fori_loop(0, n_groups, grp_body, 0)

        gather_copies = {0: fire_gathers(0, bufs[0], gsems[0])}
        out_copies = {}
        for k in range(n_chunks):
            p = k % 2
            if k + 1 < n_chunks:
                if k - 1 >= 0:
                    out_copies[k - 1].wait()
                gather_copies[k + 1] = fire_gathers(
                    k + 1, bufs[(k + 1) % 2], gsems[(k + 1) % 2]
                )
            for c in gather_copies.pop(k):
                c.wait()
            fixup_chunk(bufs[p])
            out_start = (wid * n_chunks + k) * chunk_rows
            out_copies[k] = pltpu.async_copy(
                bufs[p], out_hbm.at[pl.ds(out_start, chunk_rows)], osems[p]
            )
        out_copies[n_chunks - 2].wait()
        out_copies[n_chunks - 1].wait()

    return sc_kernel


def kernel(embeddings, indices):
    V, D = embeddings.shape
    N = indices.size
    idx_row = 128
    chunk_rows = 640
    nw = 32
    idx3d = indices.reshape(nw, N // (nw * idx_row), idx_row).astype(jnp.int32)
    sc = _make_sc_kernel(V, D, N, chunk_rows, idx_row)
    out = sc(embeddings, idx3d)
    return out.reshape(indices.shape + (D,))


# speedup vs baseline: 3.1479x; 1.0006x over previous
import functools

import jax
import jax.numpy as jnp
from jax import lax
from jax.experimental import pallas as pl
from jax.experimental.pallas import tpu as pltpu
from jax.experimental.pallas import tpu_sc as plsc


def _make_sc_kernel(V, D, N, chunk_rows, idx_row):
    info = plsc.get_sparse_core_info()
    NC, NS, L = info.num_cores, info.num_subcores, info.num_lanes
    NW = NC * NS
    assert D % L == 0 and N % (NW * chunk_rows) == 0 and chunk_rows % idx_row == 0
    n_chunks = N // (NW * chunk_rows)
    g_streams = chunk_rows // idx_row
    n_groups = chunk_rows // L
    idx_rows_per_worker = N // (NW * idx_row)

    mesh = plsc.VectorSubcoreMesh(core_axis_name="c", subcore_axis_name="s")

    @functools.partial(
        pl.kernel,
        mesh=mesh,
        out_type=jax.ShapeDtypeStruct((N, D), jnp.float32),
        scratch_types=[
            pltpu.VMEM((idx_rows_per_worker, idx_row), jnp.int32),
            pltpu.VMEM((chunk_rows, D), jnp.float32),
            pltpu.VMEM((chunk_rows, D), jnp.float32),
            pltpu.SemaphoreType.DMA,
            pltpu.SemaphoreType.DMA,
            pltpu.SemaphoreType.DMA,
            pltpu.SemaphoreType.DMA,
        ],
        compiler_params=pltpu.CompilerParams(
            needs_layout_passes=False, use_tc_tiling_on_sc=False
        ),
    )
    def sc_kernel(emb_hbm, idx_hbm, out_hbm, idx_v, rows_a, rows_b,
                  gsem_a, gsem_b, osem_a, osem_b):
        wid = lax.axis_index("s") * NC + lax.axis_index("c")
        lane = lax.iota(jnp.int32, L)
        zero16 = jnp.zeros((L,), jnp.int32)
        pltpu.sync_copy(idx_hbm.at[wid], idx_v)

        bufs = (rows_a, rows_b)
        gsems = (gsem_a, gsem_b)
        osems = (osem_a, osem_b)

        def fire_gathers(k, buf, gsem):
            return [
                pltpu.async_copy(
                    emb_hbm.at[idx_v.at[k * g_streams + j]],
                    buf.at[pl.ds(j * idx_row, idx_row)],
                    gsem,
                )
                for j in range(g_streams)
            ]

        def fixup_chunk(buf):
            def grp_body(g, carry2):
                row_ids = g * L + lane
                acc = jnp.zeros((L,), jnp.float32)
                for col in range(1, D):
                    cvec = jnp.full((L,), col, jnp.int32)
                    v = plsc.load_gather(buf, [row_ids, cvec])
                    acc = acc + v * v
                x = jnp.maximum(acc + 1.0, 1.0 + 1e-12)
                i = lax.bitcast_convert_type(x, jnp.int32)
                i = 0x5F3759DF - lax.shift_right_arithmetic(i, 1)
                y = lax.bitcast_convert_type(i, jnp.float32)
                y = y * (1.5 - 0.5 * x * y * y)
                y = y * (1.5 - 0.5 * x * y * y)
                y = y * (1.5 - 0.5 * x * y * y)
                x0 = x * y
                plsc.store_scatter(buf, [row_ids, zero16], x0)
                return carry2

            lax.fori_loop(0, n_groups, grp_body, 0)

        gather_copies = {0: fire_gathers(0, bufs[0], gsems[0])}
        out_copies = {}
        for k in range(n_chunks):
            p = k % 2
            if k + 1 < n_chunks:
                if k - 1 >= 0:
                    out_copies[k - 1].wait()
                gather_copies[k + 1] = fire_gathers(
                    k + 1, bufs[(k + 1) % 2], gsems[(k + 1) % 2]
                )
            with jax.named_scope("gather_wait"):
                for c in gather_copies.pop(k):
                    c.wait()
            with jax.named_scope("fixup"):
                fixup_chunk(bufs[p])
            out_start = (wid * n_chunks + k) * chunk_rows
            out_copies[k] = pltpu.async_copy(
                bufs[p], out_hbm.at[pl.ds(out_start, chunk_rows)], osems[p]
            )
        out_copies[n_chunks - 2].wait()
        out_copies[n_chunks - 1].wait()

    return sc_kernel


def kernel(embeddings, indices):
    V, D = embeddings.shape
    N = indices.size
    idx_row = 128
    chunk_rows = 640
    nw = 32
    idx3d = indices.reshape(nw, N // (nw * idx_row), idx_row).astype(jnp.int32)
    sc = _make_sc_kernel(V, D, N, chunk_rows, idx_row)
    out = sc(embeddings, idx3d)
    return out.reshape(indices.shape + (D,))


# speedup vs baseline: 3.1602x; 1.0039x over previous
import functools

import jax
import jax.numpy as jnp
from jax import lax
from jax.experimental import pallas as pl
from jax.experimental.pallas import tpu as pltpu
from jax.experimental.pallas import tpu_sc as plsc


def _make_sc_kernel(V, D, N, chunk_rows, idx_row):
    info = plsc.get_sparse_core_info()
    NC, NS, L = info.num_cores, info.num_subcores, info.num_lanes
    NW = NC * NS
    assert D % L == 0 and N % (NW * chunk_rows) == 0 and chunk_rows % idx_row == 0
    n_chunks = N // (NW * chunk_rows)
    g_streams = chunk_rows // idx_row
    n_groups = chunk_rows // L
    idx_rows_per_worker = N // (NW * idx_row)

    mesh = plsc.VectorSubcoreMesh(core_axis_name="c", subcore_axis_name="s")

    @functools.partial(
        pl.kernel,
        mesh=mesh,
        out_type=jax.ShapeDtypeStruct((N, D), jnp.float32),
        scratch_types=[
            pltpu.VMEM((idx_rows_per_worker, idx_row), jnp.int32),
            pltpu.VMEM((chunk_rows, D), jnp.float32),
            pltpu.VMEM((chunk_rows, D), jnp.float32),
            pltpu.SemaphoreType.DMA,
            pltpu.SemaphoreType.DMA,
            pltpu.SemaphoreType.DMA,
            pltpu.SemaphoreType.DMA,
        ],
        compiler_params=pltpu.CompilerParams(
            needs_layout_passes=False, use_tc_tiling_on_sc=False
        ),
    )
    def sc_kernel(emb_hbm, idx_hbm, out_hbm, idx_v, rows_a, rows_b,
                  gsem_a, gsem_b, osem_a, osem_b):
        wid = lax.axis_index("s") * NC + lax.axis_index("c")
        lane = lax.iota(jnp.int32, L)
        zero16 = jnp.zeros((L,), jnp.int32)
        pltpu.sync_copy(idx_hbm.at[wid], idx_v)

        bufs = (rows_a, rows_b)
        gsems = (gsem_a, gsem_b)
        osems = (osem_a, osem_b)

        def fire_gathers(k, buf, gsem):
            return [
                pltpu.async_copy(
                    emb_hbm.at[idx_v.at[k * g_streams + j]],
                    buf.at[pl.ds(j * idx_row, idx_row)],
                    gsem,
                )
                for j in range(g_streams)
            ]

        def fixup_chunk(buf):
            n_acc = 8

            @plsc.parallel_loop(0, n_groups, unroll=2)
            def grp_body(g):
                row_ids = g * L + lane
                accs = [jnp.zeros((L,), jnp.float32) for _ in range(n_acc)]
                for ci, col in enumerate(range(1, D)):
                    cvec = jnp.full((L,), col, jnp.int32)
                    v = plsc.load_gather(buf, [row_ids, cvec])
                    accs[ci % n_acc] = accs[ci % n_acc] + v * v
                while len(accs) > 1:
                    accs = [a + b for a, b in zip(accs[0::2], accs[1::2])]
                x = jnp.maximum(accs[0] + 1.0, 1.0 + 1e-12)
                i = lax.bitcast_convert_type(x, jnp.int32)
                i = 0x5F3759DF - lax.shift_right_arithmetic(i, 1)
                y = lax.bitcast_convert_type(i, jnp.float32)
                y = y * (1.5 - 0.5 * x * y * y)
                y = y * (1.5 - 0.5 * x * y * y)
                y = y * (1.5 - 0.5 * x * y * y)
                x0 = x * y
                plsc.store_scatter(buf, [row_ids, zero16], x0)

        gather_copies = {0: fire_gathers(0, bufs[0], gsems[0])}
        out_copies = {}
        for k in range(n_chunks):
            p = k % 2
            if k + 1 < n_chunks:
                if k - 1 >= 0:
                    out_copies[k - 1].wait()
                gather_copies[k + 1] = fire_gathers(
                    k + 1, bufs[(k + 1) % 2], gsems[(k + 1) % 2]
                )
            with jax.named_scope("gather_wait"):
                for c in gather_copies.pop(k):
                    c.wait()
            with jax.named_scope("fixup"):
                fixup_chunk(bufs[p])
            out_start = (wid * n_chunks + k) * chunk_rows
            out_copies[k] = pltpu.async_copy(
                bufs[p], out_hbm.at[pl.ds(out_start, chunk_rows)], osems[p]
            )
        out_copies[n_chunks - 2].wait()
        out_copies[n_chunks - 1].wait()

    return sc_kernel


def kernel(embeddings, indices):
    V, D = embeddings.shape
    N = indices.size
    idx_row = 128
    chunk_rows = 640
    nw = 32
    idx3d = indices.reshape(nw, N // (nw * idx_row), idx_row).astype(jnp.int32)
    sc = _make_sc_kernel(V, D, N, chunk_rows, idx_row)
    out = sc(embeddings, idx3d)
    return out.reshape(indices.shape + (D,))


# speedup vs baseline: 4.2770x; 1.3534x over previous
import functools

import jax
import jax.numpy as jnp
from jax import lax
from jax.experimental import pallas as pl
from jax.experimental.pallas import tpu as pltpu
from jax.experimental.pallas import tpu_sc as plsc


def _make_sc_kernel(V, D, N, chunk_rows, idx_row):
    info = plsc.get_sparse_core_info()
    NC, NS, L = info.num_cores, info.num_subcores, info.num_lanes
    NW = NC * NS
    assert D % L == 0 and N % (NW * chunk_rows) == 0 and chunk_rows % idx_row == 0
    n_chunks = N // (NW * chunk_rows)
    g_streams = chunk_rows // idx_row
    n_groups = chunk_rows // L
    idx_rows_per_worker = N // (NW * idx_row)

    mesh = plsc.VectorSubcoreMesh(core_axis_name="c", subcore_axis_name="s")

    @functools.partial(
        pl.kernel,
        mesh=mesh,
        out_type=jax.ShapeDtypeStruct((N, D), jnp.float32),
        scratch_types=[
            pltpu.VMEM((idx_rows_per_worker, idx_row), jnp.int32),
            pltpu.VMEM((chunk_rows, D), jnp.float32),
            pltpu.VMEM((chunk_rows, D), jnp.float32),
            pltpu.SemaphoreType.DMA,
            pltpu.SemaphoreType.DMA,
            pltpu.SemaphoreType.DMA,
            pltpu.SemaphoreType.DMA,
        ],
        compiler_params=pltpu.CompilerParams(
            needs_layout_passes=False, use_tc_tiling_on_sc=False
        ),
    )
    def sc_kernel(emb_hbm, idx_hbm, out_hbm, idx_v, rows_a, rows_b,
                  gsem_a, gsem_b, osem_a, osem_b):
        wid = lax.axis_index("s") * NC + lax.axis_index("c")
        lane = lax.iota(jnp.int32, L)
        zero16 = jnp.zeros((L,), jnp.int32)
        pltpu.sync_copy(idx_hbm.at[wid], idx_v)

        bufs = (rows_a, rows_b)
        gsems = (gsem_a, gsem_b)
        osems = (osem_a, osem_b)

        def fire_gathers(k, buf, gsem):
            return [
                pltpu.async_copy(
                    emb_hbm.at[idx_v.at[k * g_streams + j]],
                    buf.at[pl.ds(j * idx_row, idx_row)],
                    gsem,
                )
                for j in range(g_streams)
            ]

        def fixup_chunk(buf):
            n_acc = 8
            mask_d = D - 1
            diags = [(lane + s) & mask_d for s in range(D)]

            @plsc.parallel_loop(0, n_groups)
            def grp_body(g):
                row_ids = g * L + lane
                accs = [jnp.zeros((L,), jnp.float32) for _ in range(n_acc)]
                for s in range(D):
                    v = plsc.load_gather(buf, [row_ids, diags[s]])
                    accs[s % n_acc] = accs[s % n_acc] + v * v
                v0 = plsc.load_gather(buf, [row_ids, zero16])
                while len(accs) > 1:
                    accs = [a + b for a, b in zip(accs[0::2], accs[1::2])]
                x = jnp.maximum(accs[0] - v0 * v0 + 1.0, 1.0 + 1e-12)
                i = lax.bitcast_convert_type(x, jnp.int32)
                i = 0x5F3759DF - lax.shift_right_arithmetic(i, 1)
                y = lax.bitcast_convert_type(i, jnp.float32)
                y = y * (1.5 - 0.5 * x * y * y)
                y = y * (1.5 - 0.5 * x * y * y)
                y = y * (1.5 - 0.5 * x * y * y)
                x0 = x * y
                plsc.store_scatter(buf, [row_ids, zero16], x0)

        gather_copies = {0: fire_gathers(0, bufs[0], gsems[0])}
        out_copies = {}
        for k in range(n_chunks):
            p = k % 2
            if k + 1 < n_chunks:
                if k - 1 >= 0:
                    out_copies[k - 1].wait()
                gather_copies[k + 1] = fire_gathers(
                    k + 1, bufs[(k + 1) % 2], gsems[(k + 1) % 2]
                )
            with jax.named_scope("gather_wait"):
                for c in gather_copies.pop(k):
                    c.wait()
            with jax.named_scope("fixup"):
                fixup_chunk(bufs[p])
            out_start = (wid * n_chunks + k) * chunk_rows
            out_copies[k] = pltpu.async_copy(
                bufs[p], out_hbm.at[pl.ds(out_start, chunk_rows)], osems[p]
            )
        out_copies[n_chunks - 2].wait()
        out_copies[n_chunks - 1].wait()

    return sc_kernel


def kernel(embeddings, indices):
    V, D = embeddings.shape
    N = indices.size
    idx_row = 128
    chunk_rows = 640
    nw = 32
    idx3d = indices.reshape(nw, N // (nw * idx_row), idx_row).astype(jnp.int32)
    sc = _make_sc_kernel(V, D, N, chunk_rows, idx_row)
    out = sc(embeddings, idx3d)
    return out.reshape(indices.shape + (D,))
